# baseline (speedup 1.0000x reference)
import functools

import jax
import jax.numpy as jnp
from jax import lax
from jax.experimental import pallas as pl
from jax.experimental.pallas import tpu as pltpu
from jax.experimental.pallas import tpu_sc as plsc

CTX = 8192
DIM = 1024
S_ROWS = 2048
TC_BLK = 2048
C_BLK = 512
ROWS_C = 24
NBUF = 3


def _chunk_schedule(total, step):
    chunks = []
    r = 0
    while r < total:
        c = min(step, total - r)
        chunks.append((r, c))
        r += c
    return chunks


def _sc_copy(table):
    info = plsc.get_sparse_core_info()
    nw = info.num_cores * info.num_subcores
    rows_per_w = S_ROWS // nw
    chunks = _chunk_schedule(rows_per_w, ROWS_C)

    mesh = plsc.VectorSubcoreMesh(core_axis_name="c", subcore_axis_name="s")

    @functools.partial(
        pl.kernel,
        mesh=mesh,
        out_type=jax.ShapeDtypeStruct((S_ROWS, DIM), jnp.float32),
        scratch_types=(
            [pltpu.VMEM((ROWS_C, DIM), jnp.float32)] * NBUF
            + [pltpu.SemaphoreType.DMA] * (2 * NBUF)
        ),
    )
    def copy_kernel(table_hbm, out_hbm, *scratch):
        bufs = scratch[:NBUF]
        rsems = scratch[NBUF : 2 * NBUF]
        wsems = scratch[2 * NBUF :]
        wid = lax.axis_index("s") * info.num_cores + lax.axis_index("c")
        base = wid * rows_per_w

        def start_read(g):
            off, cn = chunks[g]
            b = g % NBUF
            return pltpu.async_copy(
                table_hbm.at[pl.ds(base + off, cn)],
                bufs[b].at[pl.ds(0, cn)],
                rsems[b],
            )

        reads = [None] * NBUF
        writes = [None] * NBUF
        reads[0] = start_read(0)
        for g in range(len(chunks)):
            b = g % NBUF
            off, cn = chunks[g]
            if g + 1 < len(chunks):
                nb = (g + 1) % NBUF
                if writes[nb] is not None:
                    writes[nb].wait()
                    writes[nb] = None
                reads[nb] = start_read(g + 1)
            reads[b].wait()
            writes[b] = pltpu.async_copy(
                bufs[b].at[pl.ds(0, cn)],
                out_hbm.at[pl.ds(base + off, cn)],
                wsems[b],
            )
        for w in writes:
            if w is not None:
                w.wait()

    return copy_kernel(table)


def _tc_body(x_ref, o_ref):
    o_ref[...] = x_ref[...]


def _tc_copy_tail(table):
    n_blk = (CTX - S_ROWS) // TC_BLK
    off = S_ROWS // TC_BLK
    return pl.pallas_call(
        _tc_body,
        grid=(n_blk,),
        in_specs=[pl.BlockSpec((TC_BLK, DIM), lambda i: (i + off, 0))],
        out_specs=pl.BlockSpec((TC_BLK, DIM), lambda i: (i + off, 0)),
        out_shape=jax.ShapeDtypeStruct((CTX, DIM), jnp.float32),
    )(table)


def _merge_body(full_ref, part_ref, o_ref):
    del full_ref
    o_ref[...] = part_ref[...]


def _merge(tc_out, sc_part):
    return pl.pallas_call(
        _merge_body,
        grid=(S_ROWS // C_BLK,),
        in_specs=[
            pl.BlockSpec(memory_space=pl.ANY),
            pl.BlockSpec((C_BLK, DIM), lambda i: (i, 0)),
        ],
        out_specs=pl.BlockSpec((C_BLK, DIM), lambda i: (i, 0)),
        out_shape=jax.ShapeDtypeStruct((CTX, DIM), jnp.float32),
        input_output_aliases={0: 0},
    )(tc_out, sc_part)


@jax.jit
def _lookup(table):
    sc_part = _sc_copy(table)
    tc_out = _tc_copy_tail(table)
    return _merge(tc_out, sc_part)


def kernel(table):
    return _lookup(table)


# speedup vs baseline: 1.0332x; 1.0332x over previous
import functools

import jax
import jax.numpy as jnp
from jax import lax
from jax.experimental import pallas as pl
from jax.experimental.pallas import tpu as pltpu
from jax.experimental.pallas import tpu_sc as plsc

CTX = 8192
DIM = 1024
S_ROWS = 1024
TC_BLK = 1024
C_BLK = 1024
ROWS_C = 24
NBUF = 3


def _chunk_schedule(total, step):
    chunks = []
    r = 0
    while r < total:
        c = min(step, total - r)
        chunks.append((r, c))
        r += c
    return chunks


def _sc_copy(table):
    info = plsc.get_sparse_core_info()
    nw = info.num_cores * info.num_subcores
    rows_per_w = S_ROWS // nw
    chunks = _chunk_schedule(rows_per_w, ROWS_C)

    mesh = plsc.VectorSubcoreMesh(core_axis_name="c", subcore_axis_name="s")

    @functools.partial(
        pl.kernel,
        mesh=mesh,
        out_type=jax.ShapeDtypeStruct((S_ROWS, DIM), jnp.float32),
        scratch_types=(
            [pltpu.VMEM((ROWS_C, DIM), jnp.float32)] * NBUF
            + [pltpu.SemaphoreType.DMA] * (2 * NBUF)
        ),
    )
    def copy_kernel(table_hbm, out_hbm, *scratch):
        bufs = scratch[:NBUF]
        rsems = scratch[NBUF : 2 * NBUF]
        wsems = scratch[2 * NBUF :]
        wid = lax.axis_index("s") * info.num_cores + lax.axis_index("c")
        base = wid * rows_per_w

        def start_read(g):
            off, cn = chunks[g]
            b = g % NBUF
            return pltpu.async_copy(
                table_hbm.at[pl.ds(base + off, cn)],
                bufs[b].at[pl.ds(0, cn)],
                rsems[b],
            )

        reads = [None] * NBUF
        writes = [None] * NBUF
        reads[0] = start_read(0)
        for g in range(len(chunks)):
            b = g % NBUF
            off, cn = chunks[g]
            if g + 1 < len(chunks):
                nb = (g + 1) % NBUF
                if writes[nb] is not None:
                    writes[nb].wait()
                    writes[nb] = None
                reads[nb] = start_read(g + 1)
            reads[b].wait()
            writes[b] = pltpu.async_copy(
                bufs[b].at[pl.ds(0, cn)],
                out_hbm.at[pl.ds(base + off, cn)],
                wsems[b],
            )
        for w in writes:
            if w is not None:
                w.wait()

    return copy_kernel(table)


def _tc_body(x_ref, o_ref):
    o_ref[...] = x_ref[...]


def _tc_copy_tail(table):
    n_blk = (CTX - S_ROWS) // TC_BLK
    off = S_ROWS // TC_BLK
    return pl.pallas_call(
        _tc_body,
        grid=(n_blk,),
        in_specs=[pl.BlockSpec((TC_BLK, DIM), lambda i: (i + off, 0))],
        out_specs=pl.BlockSpec((TC_BLK, DIM), lambda i: (i + off, 0)),
        out_shape=jax.ShapeDtypeStruct((CTX, DIM), jnp.float32),
    )(table)


def _merge_body(full_ref, part_ref, o_ref):
    del full_ref
    o_ref[...] = part_ref[...]


def _merge(tc_out, sc_part):
    return pl.pallas_call(
        _merge_body,
        grid=(S_ROWS // C_BLK,),
        in_specs=[
            pl.BlockSpec(memory_space=pl.ANY),
            pl.BlockSpec((C_BLK, DIM), lambda i: (i, 0)),
        ],
        out_specs=pl.BlockSpec((C_BLK, DIM), lambda i: (i, 0)),
        out_shape=jax.ShapeDtypeStruct((CTX, DIM), jnp.float32),
        input_output_aliases={0: 0},
    )(tc_out, sc_part)


@jax.jit
def _lookup(table):
    sc_part = _sc_copy(table)
    tc_out = _tc_copy_tail(table)
    return _merge(tc_out, sc_part)


def kernel(table):
    return _lookup(table)
